# baseline (speedup 1.0000x reference)
import functools

import jax
import jax.numpy as jnp
from jax import lax
from jax.experimental import pallas as pl
from jax.experimental.pallas import tpu as pltpu
from jax.experimental.pallas import tpu_sc as plsc

B, C, H, W = 4, 96, 384, 384
HW = H * W
N = B * HW
NC, NS = 2, 16
R = 32
BAND = R * W
NPASS = H // (R * NC)
PXT = HW // NS
NB = PXT // 16
ROWS_T = PXT // W
CB = W // 16
SLICE = BAND // NS
ZR = 16
PC = 128
TRASH = BAND


def _weight_tc(src, fx, fy):
  def body(src_ref, fx_ref, fy_ref, w4_ref, dn_ref):
    i = pl.program_id(0)
    y_img = i % H
    b = i // H
    gx = lax.broadcasted_iota(jnp.int32, (W, 1), 0).astype(jnp.float32)
    x = gx + fx_ref[...]
    y = y_img.astype(jnp.float32) + fy_ref[...]
    x = jnp.clip(x, -2.0, 385.0)
    y = jnp.clip(y, -2.0, 385.0)
    x_f = jnp.floor(x)
    y_f = jnp.floor(y)
    valid = ((x_f >= 0) & (x_f <= W - 2) & (y_f >= 0) & (y_f <= H - 2))
    dx = x - x_f
    dy = y - y_f
    omdx = 1.0 - dx
    omdy = 1.0 - dy
    zf = jnp.zeros((W, 1), jnp.float32)
    xfi = jnp.clip(x_f.astype(jnp.int32), 0, W - 2)
    yfi = jnp.clip(y_f.astype(jnp.int32), 0, H - 2)
    dn_ref[...] = b * HW + yfi * W + xfi
    srcv = src_ref[...]
    zpad = jnp.zeros((W, PC - C), jnp.float32)
    for k, wk in enumerate((omdx * omdy, dx * omdy, omdx * dy, dx * dy)):
      w4_ref[k, :, 0:C] = srcv * jnp.where(valid, wk, zf)
      w4_ref[k, :, C:PC] = zpad

  return pl.pallas_call(
      body,
      grid=(B * H,),
      in_specs=[
          pl.BlockSpec((W, C), lambda i: (i, 0)),
          pl.BlockSpec((W, 1), lambda i: (i, 0)),
          pl.BlockSpec((W, 1), lambda i: (i, 0)),
      ],
      out_specs=[
          pl.BlockSpec((4, W, PC), lambda i: (0, i, 0)),
          pl.BlockSpec((W, 1), lambda i: (i, 0)),
      ],
      out_shape=[
          jax.ShapeDtypeStruct((4, N, PC), jnp.float32),
          jax.ShapeDtypeStruct((N, 1), jnp.int32),
      ],
  )(src, fx, fy)


def _scatter_sc(w4, dn, fy):
  mesh = plsc.VectorSubcoreMesh(core_axis_name="c", subcore_axis_name="s")

  @functools.partial(
      pl.kernel,
      out_type=jax.ShapeDtypeStruct((N, PC), jnp.float32),
      mesh=mesh,
      scratch_types=dict(
          fyv=pltpu.VMEM((PXT,), jnp.float32),
          dnv=pltpu.VMEM((PXT,), jnp.int32),
          rbuf0=pltpu.VMEM((16, PC), jnp.float32),
          rbuf1=pltpu.VMEM((16, PC), jnp.float32),
          rbuf2=pltpu.VMEM((16, PC), jnp.float32),
          rbuf3=pltpu.VMEM((16, PC), jnp.float32),
          zbuf=pltpu.VMEM((ZR, PC), jnp.float32),
          ymm=pltpu.SMEM((2 * NB,), jnp.float32),
          acc=pltpu.VMEM_SHARED((BAND + 8, PC), jnp.float32),
          gsem=pltpu.SemaphoreType.DMA,
          ssem=pltpu.SemaphoreType.DMA,
      ),
  )
  def warp(w4_hbm, dn_hbm, fy_hbm, out_hbm, *, fyv, dnv, rbuf0, rbuf1,
           rbuf2, rbuf3, zbuf, ymm, acc, gsem, ssem):
    c = lax.axis_index("c")
    s = lax.axis_index("s")
    zeros16f = jnp.zeros((16,), jnp.float32)
    rbufs = (rbuf0, rbuf1, rbuf2, rbuf3)

    def _z(i, _):
      for k in range(PC // 16):
        zbuf[i, pl.ds(k * 16, 16)] = zeros16f
      return 0
    lax.fori_loop(0, ZR, _z, 0)

    def batch_body(b, _):
      base_b = b * HW
      pltpu.sync_copy(fy_hbm.at[pl.ds(base_b + s * PXT, PXT)], fyv)
      pltpu.sync_copy(dn_hbm.at[pl.ds(base_b + s * PXT, PXT)], dnv)

      def mm_body(i, _):
        v = fyv[pl.ds(16 * i, 16)]
        lo = v[0]
        hi = v[0]
        for l in range(1, 16):
          v_l = v[l]
          lo = jnp.minimum(lo, v_l)
          hi = jnp.maximum(hi, v_l)
        ymm[2 * i] = lo
        ymm[2 * i + 1] = hi
        return 0
      lax.fori_loop(0, NB, mm_body, 0)

      def pass_body(p, _):
        row0 = (p * NC + c) * R
        row0_f = row0.astype(jnp.float32)
        base_row = base_b + row0 * W
        for j in range(SLICE // ZR):
          pltpu.sync_copy(zbuf, acc.at[pl.ds(s * SLICE + j * ZR, ZR)])
        plsc.subcore_barrier()

        def blk_body(i, _):
          rr = i // CB
          gy_f = (s * ROWS_T + rr).astype(jnp.float32)
          ymin = gy_f + ymm[2 * i]
          ymax = gy_f + ymm[2 * i + 1]
          hit = (ymax >= row0_f - 1.0) & (ymin < row0_f + R)

          @pl.when(hit)
          def _do():
            off = i * 16
            g0 = base_b + s * PXT + off
            hs = [pltpu.async_copy(w4_hbm.at[pl.ds(k * N + g0, 16)],
                                   rbufs[k], gsem) for k in range(4)]
            dvec = dnv[pl.ds(off, 16)]
            rel0 = dvec - base_row
            dsts = (rel0, rel0 + 1, rel0 + W, rel0 + W + 1)
            ti = jnp.full((16,), TRASH, jnp.int32)
            dsts = [jnp.where((d >= 0) & (d < BAND), d, ti) for d in dsts]
            for h in hs:
              h.wait()
            sc = [pltpu.async_copy(rbufs[k], acc.at[dsts[k]], ssem, add=True)
                  for k in range(4)]
            for h in sc:
              h.wait()

          return 0

        lax.fori_loop(0, NB, blk_body, 0)
        plsc.subcore_barrier()
        out_base = base_b + row0 * W + s * SLICE
        pltpu.sync_copy(acc.at[pl.ds(s * SLICE, SLICE)],
                        out_hbm.at[pl.ds(out_base, SLICE)])
        return 0

      lax.fori_loop(0, NPASS, pass_body, 0)
      return 0

    lax.fori_loop(0, B, batch_body, 0)

  return warp(w4, dn, fy)


def kernel(im0, flow):
  src = jnp.transpose(im0, (0, 2, 3, 1)).reshape(N, C)
  fx = flow[..., 0].reshape(N, 1)
  fy = flow[..., 1].reshape(N, 1)
  w4, dn = _weight_tc(src, fx, fy)
  out = _scatter_sc(w4.reshape(4 * N, PC), dn.reshape(N), fy.reshape(N))
  return out.reshape(B, H, W, PC)[..., :C].transpose(0, 3, 1, 2)


# speedup vs baseline: 1.2144x; 1.2144x over previous
import functools

import jax
import jax.numpy as jnp
from jax import lax
from jax.experimental import pallas as pl
from jax.experimental.pallas import tpu as pltpu
from jax.experimental.pallas import tpu_sc as plsc

B, C, H, W = 4, 96, 384, 384
HW = H * W
N = B * HW
NC, NS = 2, 16
R = 32
BAND = R * W
NPASS = H // (R * NC)
PXT = HW // NS
ROWS_T = PXT // W
NCH = W // 16
SLICE = BAND // NS
ZR = 16
PC = 128
TRASH = BAND


def _weight_tc(src, fx, fy):
  def body(src_ref, fx_ref, fy_ref, w4_ref, dq_ref):
    i = pl.program_id(0)
    y_img = i % H
    b = i // H
    gx = lax.broadcasted_iota(jnp.int32, (W, 1), 0).astype(jnp.float32)
    x = gx + fx_ref[...]
    y = y_img.astype(jnp.float32) + fy_ref[...]
    x = jnp.clip(x, -2.0, 385.0)
    y = jnp.clip(y, -2.0, 385.0)
    x_f = jnp.floor(x)
    y_f = jnp.floor(y)
    valid = ((x_f >= 0) & (x_f <= W - 2) & (y_f >= 0) & (y_f <= H - 2))
    dx = x - x_f
    dy = y - y_f
    omdx = 1.0 - dx
    omdy = 1.0 - dy
    zf = jnp.zeros((W, 1), jnp.float32)
    xfi = jnp.clip(x_f.astype(jnp.int32), 0, W - 2)
    yfi = jnp.clip(y_f.astype(jnp.int32), 0, H - 2)
    dnw = b * HW + yfi * W + xfi
    srcv = src_ref[...]
    zpad = jnp.zeros((W, PC - C), jnp.float32)
    wks = (omdx * omdy, dx * omdy, omdx * dy, dx * dy)
    offs = (0, 1, W, W + 1)
    for k in range(4):
      w4_ref[:, k, 0:C] = srcv * jnp.where(valid, wks[k], zf)
      w4_ref[:, k, C:PC] = zpad
      dq_ref[:, k:k + 1] = dnw + offs[k]

  return pl.pallas_call(
      body,
      grid=(B * H,),
      in_specs=[
          pl.BlockSpec((W, C), lambda i: (i, 0)),
          pl.BlockSpec((W, 1), lambda i: (i, 0)),
          pl.BlockSpec((W, 1), lambda i: (i, 0)),
      ],
      out_specs=[
          pl.BlockSpec((W, 4, PC), lambda i: (i, 0, 0)),
          pl.BlockSpec((W, 4), lambda i: (i, 0)),
      ],
      out_shape=[
          jax.ShapeDtypeStruct((N, 4, PC), jnp.float32),
          jax.ShapeDtypeStruct((N, 4), jnp.int32),
      ],
  )(src, fx, fy)


def _scatter_sc(w4, dq, fy):
  mesh = plsc.VectorSubcoreMesh(core_axis_name="c", subcore_axis_name="s")

  @functools.partial(
      pl.kernel,
      out_type=jax.ShapeDtypeStruct((N, PC), jnp.float32),
      mesh=mesh,
      scratch_types=dict(
          fyv=pltpu.VMEM((PXT,), jnp.float32),
          rb0=pltpu.VMEM((64, PC), jnp.float32),
          rb1=pltpu.VMEM((64, PC), jnp.float32),
          dqr0=pltpu.VMEM((64,), jnp.int32),
          dqr1=pltpu.VMEM((64,), jnp.int32),
          dqv0=pltpu.VMEM((64,), jnp.int32),
          dqv1=pltpu.VMEM((64,), jnp.int32),
          zbuf=pltpu.VMEM((ZR, PC), jnp.float32),
          ymm=pltpu.SMEM((2 * ROWS_T,), jnp.float32),
          acc=pltpu.VMEM_SHARED((BAND + 8, PC), jnp.float32),
          gsem=pltpu.SemaphoreType.DMA,
          ssem=pltpu.SemaphoreType.DMA,
      ),
  )
  def warp(w4_hbm, dq_hbm, fy_hbm, out_hbm, *, fyv, rb0, rb1, dqr0, dqr1,
           dqv0, dqv1, zbuf, ymm, acc, gsem, ssem):
    c = lax.axis_index("c")
    s = lax.axis_index("s")
    zeros16f = jnp.zeros((16,), jnp.float32)
    rbufs = (rb0, rb1)
    dqrs = (dqr0, dqr1)
    dqvs = (dqv0, dqv1)

    def _z(i, _):
      for k in range(PC // 16):
        zbuf[i, pl.ds(k * 16, 16)] = zeros16f
      return 0
    lax.fori_loop(0, ZR, _z, 0)

    def batch_body(b, _):
      base_b = b * HW
      pltpu.sync_copy(fy_hbm.at[pl.ds(base_b + s * PXT, PXT)], fyv)

      def mm_body(rr, _):
        lo_v = fyv[pl.ds(rr * W, 16)]
        hi_v = lo_v
        for j in range(1, NCH):
          v = fyv[pl.ds(rr * W + j * 16, 16)]
          lo_v = jnp.minimum(lo_v, v)
          hi_v = jnp.maximum(hi_v, v)
        lo = lo_v[0]
        hi = hi_v[0]
        for l in range(1, 16):
          lo = jnp.minimum(lo, lo_v[l])
          hi = jnp.maximum(hi, hi_v[l])
        ymm[2 * rr] = lo
        ymm[2 * rr + 1] = hi
        return 0
      lax.fori_loop(0, ROWS_T, mm_body, 0)

      def pass_body(p, _):
        row0 = (p * NC + c) * R
        row0_f = row0.astype(jnp.float32)
        base_row = base_b + row0 * W
        for j in range(SLICE // ZR):
          pltpu.sync_copy(zbuf, acc.at[pl.ds(s * SLICE + j * ZR, ZR)])
        plsc.subcore_barrier()

        def row_body(rr, _):
          gy_f = (s * ROWS_T + rr).astype(jnp.float32)
          hit = ((gy_f + ymm[2 * rr + 1] >= row0_f - 1.0)
                 & (gy_f + ymm[2 * rr] < row0_f + R))

          @pl.when(hit)
          def _do():
            g0 = base_b + s * PXT + rr * W

            def issue_read(ch):
              bb = ch % 2
              at = pl.ds(4 * (g0 + ch * 16), 64)
              pltpu.sync_copy(dq_hbm.at[at], dqrs[bb])
              return pltpu.async_copy(w4_hbm.at[at], rbufs[bb], gsem)

            ghs = [None] * NCH
            shs = [None] * NCH
            ghs[0] = issue_read(0)
            for ch in range(NCH):
              bb = ch % 2
              if ch + 1 < NCH:
                if ch - 1 >= 0:
                  shs[ch - 1].wait()
                ghs[ch + 1] = issue_read(ch + 1)
              ti = jnp.full((16,), TRASH, jnp.int32)
              for j in range(4):
                v = dqrs[bb][pl.ds(j * 16, 16)]
                rel = v - base_row
                dqvs[bb][pl.ds(j * 16, 16)] = jnp.where(
                    (rel >= 0) & (rel < BAND), rel, ti)
              ghs[ch].wait()
              shs[ch] = pltpu.async_copy(rbufs[bb], acc.at[dqvs[bb]], ssem,
                                         add=True)
            shs[NCH - 2].wait()
            shs[NCH - 1].wait()

          return 0

        lax.fori_loop(0, ROWS_T, row_body, 0)
        plsc.subcore_barrier()
        out_base = base_b + row0 * W + s * SLICE
        pltpu.sync_copy(acc.at[pl.ds(s * SLICE, SLICE)],
                        out_hbm.at[pl.ds(out_base, SLICE)])
        return 0

      lax.fori_loop(0, NPASS, pass_body, 0)
      return 0

    lax.fori_loop(0, B, batch_body, 0)

  return warp(w4, dq, fy)


def kernel(im0, flow):
  src = jnp.transpose(im0, (0, 2, 3, 1)).reshape(N, C)
  fx = flow[..., 0].reshape(N, 1)
  fy = flow[..., 1].reshape(N, 1)
  w4, dq = _weight_tc(src, fx, fy)
  out = _scatter_sc(w4.reshape(4 * N, PC), dq.reshape(4 * N), fy.reshape(N))
  return out.reshape(B, H, W, PC)[..., :C].transpose(0, 3, 1, 2)


# speedup vs baseline: 1.4585x; 1.2010x over previous
import functools

import jax
import jax.numpy as jnp
from jax import lax
from jax.experimental import pallas as pl
from jax.experimental.pallas import tpu as pltpu
from jax.experimental.pallas import tpu_sc as plsc

B, C, H, W = 4, 96, 384, 384
HW = H * W
N = B * HW
NC, NS = 2, 16
R = 32
BAND = R * W
NPASS = H // (R * NC)
PXT = HW // NS
ROWS_T = PXT // W
NCH = W // 16
SLICE = BAND // NS
ZR = 16
PC = 128
TRASH = BAND


def _weight_tc(src, fx, fy):
  def body(src_ref, fx_ref, fy_ref, w4_ref, dq_ref):
    i = pl.program_id(0)
    y_img = i % H
    b = i // H
    gx = lax.broadcasted_iota(jnp.int32, (W, 1), 0).astype(jnp.float32)
    x = gx + fx_ref[...]
    y = y_img.astype(jnp.float32) + fy_ref[...]
    x = jnp.clip(x, -2.0, 385.0)
    y = jnp.clip(y, -2.0, 385.0)
    x_f = jnp.floor(x)
    y_f = jnp.floor(y)
    valid = ((x_f >= 0) & (x_f <= W - 2) & (y_f >= 0) & (y_f <= H - 2))
    dx = x - x_f
    dy = y - y_f
    omdx = 1.0 - dx
    omdy = 1.0 - dy
    zf = jnp.zeros((W, 1), jnp.float32)
    xfi = jnp.clip(x_f.astype(jnp.int32), 0, W - 2)
    yfi = jnp.clip(y_f.astype(jnp.int32), 0, H - 2)
    dnw = b * HW + yfi * W + xfi
    srcv = src_ref[...]
    zpad = jnp.zeros((W, PC - C), jnp.float32)
    wks = (omdx * omdy, dx * omdy, omdx * dy, dx * dy)
    offs = (0, 1, W, W + 1)
    for k in range(4):
      w4_ref[:, k, 0:C] = srcv * jnp.where(valid, wks[k], zf)
      w4_ref[:, k, C:PC] = zpad
      dq_ref[:, k:k + 1] = dnw + offs[k]

  return pl.pallas_call(
      body,
      grid=(B * H,),
      in_specs=[
          pl.BlockSpec((W, C), lambda i: (i, 0)),
          pl.BlockSpec((W, 1), lambda i: (i, 0)),
          pl.BlockSpec((W, 1), lambda i: (i, 0)),
      ],
      out_specs=[
          pl.BlockSpec((W, 4, PC), lambda i: (i, 0, 0)),
          pl.BlockSpec((W, 4), lambda i: (i, 0)),
      ],
      out_shape=[
          jax.ShapeDtypeStruct((N, 4, PC), jnp.float32),
          jax.ShapeDtypeStruct((N, 4), jnp.int32),
      ],
  )(src, fx, fy)


def _scatter_sc(w4, dq, fy):
  mesh = plsc.VectorSubcoreMesh(core_axis_name="c", subcore_axis_name="s")

  @functools.partial(
      pl.kernel,
      out_type=jax.ShapeDtypeStruct((N, PC), jnp.float32),
      mesh=mesh,
      scratch_types=dict(
          fyv=pltpu.VMEM((PXT,), jnp.float32),
          rb0=pltpu.VMEM((64, PC), jnp.float32),
          rb1=pltpu.VMEM((64, PC), jnp.float32),
          dqraw=pltpu.VMEM((4 * W,), jnp.int32),
          dqv0=pltpu.VMEM((64,), jnp.int32),
          dqv1=pltpu.VMEM((64,), jnp.int32),
          zbuf=pltpu.VMEM((ZR, PC), jnp.float32),
          ymm=pltpu.SMEM((2 * ROWS_T,), jnp.float32),
          acc=pltpu.VMEM_SHARED((BAND + 8, PC), jnp.float32),
          gsem=pltpu.SemaphoreType.DMA,
          ssem=pltpu.SemaphoreType.DMA,
      ),
  )
  def warp(w4_hbm, dq_hbm, fy_hbm, out_hbm, *, fyv, rb0, rb1, dqraw,
           dqv0, dqv1, zbuf, ymm, acc, gsem, ssem):
    c = lax.axis_index("c")
    s = lax.axis_index("s")
    zeros16f = jnp.zeros((16,), jnp.float32)
    rbufs = (rb0, rb1)
    dqvs = (dqv0, dqv1)

    def _z(i, _):
      for k in range(PC // 16):
        zbuf[i, pl.ds(k * 16, 16)] = zeros16f
      return 0
    lax.fori_loop(0, ZR, _z, 0)

    def batch_body(b, _):
      base_b = b * HW
      pltpu.sync_copy(fy_hbm.at[pl.ds(base_b + s * PXT, PXT)], fyv)

      def mm_body(rr, _):
        lo_v = fyv[pl.ds(rr * W, 16)]
        hi_v = lo_v
        for j in range(1, NCH):
          v = fyv[pl.ds(rr * W + j * 16, 16)]
          lo_v = jnp.minimum(lo_v, v)
          hi_v = jnp.maximum(hi_v, v)
        lo = lo_v[0]
        hi = hi_v[0]
        for l in range(1, 16):
          lo = jnp.minimum(lo, lo_v[l])
          hi = jnp.maximum(hi, hi_v[l])
        ymm[2 * rr] = lo
        ymm[2 * rr + 1] = hi
        return 0
      lax.fori_loop(0, ROWS_T, mm_body, 0)

      def pass_body(p, _):
        row0 = (p * NC + c) * R
        row0_f = row0.astype(jnp.float32)
        base_row = base_b + row0 * W
        for j in range(SLICE // ZR):
          pltpu.sync_copy(zbuf, acc.at[pl.ds(s * SLICE + j * ZR, ZR)])
        plsc.subcore_barrier()

        def row_body(rr, _):
          gy_f = (s * ROWS_T + rr).astype(jnp.float32)
          hit = ((gy_f + ymm[2 * rr + 1] >= row0_f - 1.0)
                 & (gy_f + ymm[2 * rr] < row0_f + R))

          @pl.when(hit)
          def _do():
            g0 = base_b + s * PXT + rr * W
            pltpu.sync_copy(dq_hbm.at[pl.ds(4 * g0, 4 * W)], dqraw)

            def issue_read(ch):
              bb = ch % 2
              at = pl.ds(4 * (g0 + ch * 16), 64)
              return pltpu.async_copy(w4_hbm.at[at], rbufs[bb], gsem)

            ghs = [None] * NCH
            shs = [None] * NCH
            ghs[0] = issue_read(0)
            for ch in range(NCH):
              bb = ch % 2
              if ch + 1 < NCH:
                if ch - 1 >= 0:
                  shs[ch - 1].wait()
                ghs[ch + 1] = issue_read(ch + 1)
              ti = jnp.full((16,), TRASH, jnp.int32)
              for j in range(4):
                v = dqraw[pl.ds(ch * 64 + j * 16, 16)]
                rel = v - base_row
                dqvs[bb][pl.ds(j * 16, 16)] = jnp.where(
                    (rel >= 0) & (rel < BAND), rel, ti)
              ghs[ch].wait()
              shs[ch] = pltpu.async_copy(rbufs[bb], acc.at[dqvs[bb]], ssem,
                                         add=True)
            shs[NCH - 2].wait()
            shs[NCH - 1].wait()

          return 0

        lax.fori_loop(0, ROWS_T, row_body, 0)
        plsc.subcore_barrier()
        out_base = base_b + row0 * W + s * SLICE
        pltpu.sync_copy(acc.at[pl.ds(s * SLICE, SLICE)],
                        out_hbm.at[pl.ds(out_base, SLICE)])
        return 0

      lax.fori_loop(0, NPASS, pass_body, 0)
      return 0

    lax.fori_loop(0, B, batch_body, 0)

  return warp(w4, dq, fy)


def kernel(im0, flow):
  src = jnp.transpose(im0, (0, 2, 3, 1)).reshape(N, C)
  fx = flow[..., 0].reshape(N, 1)
  fy = flow[..., 1].reshape(N, 1)
  w4, dq = _weight_tc(src, fx, fy)
  out = _scatter_sc(w4.reshape(4 * N, PC), dq.reshape(4 * N), fy.reshape(N))
  return out.reshape(B, H, W, PC)[..., :C].transpose(0, 3, 1, 2)


# speedup vs baseline: 1.4959x; 1.0257x over previous
import functools

import jax
import jax.numpy as jnp
from jax import lax
from jax.experimental import pallas as pl
from jax.experimental.pallas import tpu as pltpu
from jax.experimental.pallas import tpu_sc as plsc

B, C, H, W = 4, 96, 384, 384
HW = H * W
N = B * HW
NC, NS = 2, 16
R = 32
BAND = R * W
NPASS = H // (R * NC)
PXT = HW // NS
ROWS_T = PXT // W
NCH = W // 16
SLICE = BAND // NS
ZR = 32
PC = 128
TRASH = BAND


def _weight_tc(src, fx, fy):
  def body(src_ref, fx_ref, fy_ref, w4_ref, dq_ref):
    i = pl.program_id(0)
    y_img = i % H
    b = i // H
    gx = lax.broadcasted_iota(jnp.int32, (W, 1), 0).astype(jnp.float32)
    x = gx + fx_ref[...]
    y = y_img.astype(jnp.float32) + fy_ref[...]
    x = jnp.clip(x, -2.0, 385.0)
    y = jnp.clip(y, -2.0, 385.0)
    x_f = jnp.floor(x)
    y_f = jnp.floor(y)
    valid = ((x_f >= 0) & (x_f <= W - 2) & (y_f >= 0) & (y_f <= H - 2))
    dx = x - x_f
    dy = y - y_f
    omdx = 1.0 - dx
    omdy = 1.0 - dy
    zf = jnp.zeros((W, 1), jnp.float32)
    xfi = jnp.clip(x_f.astype(jnp.int32), 0, W - 2)
    yfi = jnp.clip(y_f.astype(jnp.int32), 0, H - 2)
    dnw = b * HW + yfi * W + xfi
    srcv = src_ref[...]
    zpad = jnp.zeros((W, PC - C), jnp.float32)
    wks = (omdx * omdy, dx * omdy, omdx * dy, dx * dy)
    offs = (0, 1, W, W + 1)
    for k in range(4):
      w4_ref[:, k, 0:C] = srcv * jnp.where(valid, wks[k], zf)
      w4_ref[:, k, C:PC] = zpad
      dq_ref[:, k:k + 1] = dnw + offs[k]

  return pl.pallas_call(
      body,
      grid=(B * H,),
      in_specs=[
          pl.BlockSpec((W, C), lambda i: (i, 0)),
          pl.BlockSpec((W, 1), lambda i: (i, 0)),
          pl.BlockSpec((W, 1), lambda i: (i, 0)),
      ],
      out_specs=[
          pl.BlockSpec((W, 4, PC), lambda i: (i, 0, 0)),
          pl.BlockSpec((W, 4), lambda i: (i, 0)),
      ],
      out_shape=[
          jax.ShapeDtypeStruct((N, 4, PC), jnp.float32),
          jax.ShapeDtypeStruct((N, 4), jnp.int32),
      ],
  )(src, fx, fy)


def _scatter_sc(w4, dq, fy):
  mesh = plsc.VectorSubcoreMesh(core_axis_name="c", subcore_axis_name="s")

  @functools.partial(
      pl.kernel,
      out_type=jax.ShapeDtypeStruct((N, PC), jnp.float32),
      mesh=mesh,
      scratch_types=dict(
          fyv=pltpu.VMEM((PXT,), jnp.float32),
          rb0=pltpu.VMEM((64, PC), jnp.float32),
          rb1=pltpu.VMEM((64, PC), jnp.float32),
          dqraw=pltpu.VMEM((4 * W,), jnp.int32),
          dqv0=pltpu.VMEM((64,), jnp.int32),
          dqv1=pltpu.VMEM((64,), jnp.int32),
          zbuf=pltpu.VMEM((ZR, PC), jnp.float32),
          ymm=pltpu.SMEM((2 * ROWS_T,), jnp.float32),
          acc=pltpu.VMEM_SHARED((BAND + 8, PC), jnp.float32),
          gsem=pltpu.SemaphoreType.DMA,
          ssem=pltpu.SemaphoreType.DMA,
      ),
  )
  def warp(w4_hbm, dq_hbm, fy_hbm, out_hbm, *, fyv, rb0, rb1, dqraw,
           dqv0, dqv1, zbuf, ymm, acc, gsem, ssem):
    c = lax.axis_index("c")
    s = lax.axis_index("s")
    zeros16f = jnp.zeros((16,), jnp.float32)
    rbufs = (rb0, rb1)
    dqvs = (dqv0, dqv1)

    def _z(i, _):
      for k in range(PC // 16):
        zbuf[i, pl.ds(k * 16, 16)] = zeros16f
      return 0
    lax.fori_loop(0, ZR, _z, 0)

    def batch_body(b, _):
      base_b = b * HW
      pltpu.sync_copy(fy_hbm.at[pl.ds(base_b + s * PXT, PXT)], fyv)

      def mm_body(rr, _):
        lo_v = fyv[pl.ds(rr * W, 16)]
        hi_v = lo_v
        for j in range(1, NCH):
          v = fyv[pl.ds(rr * W + j * 16, 16)]
          lo_v = jnp.minimum(lo_v, v)
          hi_v = jnp.maximum(hi_v, v)
        lo = lo_v[0]
        hi = hi_v[0]
        for l in range(1, 16):
          lo = jnp.minimum(lo, lo_v[l])
          hi = jnp.maximum(hi, hi_v[l])
        ymm[2 * rr] = lo
        ymm[2 * rr + 1] = hi
        return 0
      lax.fori_loop(0, ROWS_T, mm_body, 0)

      def pass_body(p, _):
        row0 = (p * NC + c) * R
        row0_f = row0.astype(jnp.float32)
        base_row = base_b + row0 * W
        zhs = [pltpu.async_copy(zbuf, acc.at[pl.ds(s * SLICE + j * ZR, ZR)],
                                gsem) for j in range(SLICE // ZR)]
        for h in zhs:
          h.wait()
        plsc.subcore_barrier()

        def row_body(rr, _):
          gy_f = (s * ROWS_T + rr).astype(jnp.float32)
          hit = ((gy_f + ymm[2 * rr + 1] >= row0_f - 1.0)
                 & (gy_f + ymm[2 * rr] < row0_f + R))

          @pl.when(hit)
          def _do():
            g0 = base_b + s * PXT + rr * W
            dqh = pltpu.async_copy(dq_hbm.at[pl.ds(4 * g0, 4 * W)], dqraw,
                                   ssem)

            def issue_read(ch):
              bb = ch % 2
              at = pl.ds(4 * (g0 + ch * 16), 64)
              return pltpu.async_copy(w4_hbm.at[at], rbufs[bb], gsem)

            ghs = [None] * NCH
            shs = [None] * NCH
            ghs[0] = issue_read(0)
            dqh.wait()
            for ch in range(NCH):
              bb = ch % 2
              if ch + 1 < NCH:
                if ch - 1 >= 0:
                  shs[ch - 1].wait()
                ghs[ch + 1] = issue_read(ch + 1)
              ti = jnp.full((16,), TRASH, jnp.int32)
              for j in range(4):
                v = dqraw[pl.ds(ch * 64 + j * 16, 16)]
                rel = v - base_row
                dqvs[bb][pl.ds(j * 16, 16)] = jnp.where(
                    (rel >= 0) & (rel < BAND), rel, ti)
              ghs[ch].wait()
              shs[ch] = pltpu.async_copy(rbufs[bb], acc.at[dqvs[bb]], ssem,
                                         add=True)
            shs[NCH - 2].wait()
            shs[NCH - 1].wait()

          return 0

        lax.fori_loop(0, ROWS_T, row_body, 0)
        plsc.subcore_barrier()
        out_base = base_b + row0 * W + s * SLICE
        pltpu.sync_copy(acc.at[pl.ds(s * SLICE, SLICE)],
                        out_hbm.at[pl.ds(out_base, SLICE)])
        return 0

      lax.fori_loop(0, NPASS, pass_body, 0)
      return 0

    lax.fori_loop(0, B, batch_body, 0)

  return warp(w4, dq, fy)


def kernel(im0, flow):
  src = jnp.transpose(im0, (0, 2, 3, 1)).reshape(N, C)
  fx = flow[..., 0].reshape(N, 1)
  fy = flow[..., 1].reshape(N, 1)
  w4, dq = _weight_tc(src, fx, fy)
  out = _scatter_sc(w4.reshape(4 * N, PC), dq.reshape(4 * N), fy.reshape(N))
  return out.reshape(B, H, W, PC)[..., :C].transpose(0, 3, 1, 2)
